# baseline (speedup 1.0000x reference)
import functools

import jax
import jax.numpy as jnp
from jax import lax
from jax.experimental import pallas as pl
from jax.experimental.pallas import tpu as pltpu
from jax.experimental.pallas import tpu_sc as plsc

EMBED_DIM = 64
D_MODEL = 2 * EMBED_DIM
D_INNER = 2 * D_MODEL
HID = 2 * D_MODEL
STEPS = 4
FEW = 5
SUP_PAD = 8

_NC, _NS = 2, 16
_NW = _NC * _NS

_V = 1000001
_TCOLS = (_V + 127) // 128
_PROWS = _TCOLS * 64


@functools.lru_cache(maxsize=None)
def _make_transpose():
    mesh = plsc.VectorSubcoreMesh(core_axis_name="c", subcore_axis_name="s")

    @functools.partial(
        pl.kernel,
        mesh=mesh,
        compiler_params=pltpu.CompilerParams(use_tc_tiling_on_sc=True,
                                             needs_layout_passes=False,
                                             disable_bounds_checks=True),
        out_type=jax.ShapeDtypeStruct((_PROWS, 128), jnp.float32),
        scratch_types=[
            pltpu.VMEM((8, 8, 128), jnp.float32),
            pltpu.VMEM((8, 8, 128), jnp.float32),
            pltpu.SemaphoreType.DMA,
            pltpu.SemaphoreType.DMA,
        ],
    )
    def transpose_k(embT_hbm, p_hbm, slab, out3, sem_in, sem_out):
        wid = lax.axis_index("s") * _NC + lax.axis_index("c")
        lane = lax.iota(jnp.int32, 16)
        nchunk = (_TCOLS - wid + _NW - 1) // _NW

        i0 = [(jnp.full((16,), (16 * g) & 63, jnp.int32) + lane) >> 3
              for g in range(8)]
        i1 = [((jnp.full((16,), (16 * g) & 63, jnp.int32) + lane) >> 0) & 7
              for g in range(8)]

        def body(k, carry):
            c = wid + k * _NW
            col0 = pl.multiple_of(c * 128, 128)
            for a in range(8):
                pltpu.async_copy(
                    embT_hbm.at[pl.ds(8 * a, 8), pl.ds(col0, 128)],
                    slab.at[a], sem_in)
            for a in range(8):
                pltpu.make_async_copy(
                    embT_hbm.at[pl.ds(8 * a, 8), pl.ds(col0, 128)],
                    slab.at[a], sem_in).wait()
            for jt in range(8):
                for jr in range(8):
                    j = 8 * jt + jr
                    t0 = jnp.full((16,), 2 * j, jnp.int32)
                    t1 = jnp.full((16,), 2 * j + 1, jnp.int32)
                    for g in range(8):
                        th = t0 if g < 4 else t1
                        out3[jt, jr, pl.ds(16 * g, 16)] = plsc.load_gather(
                            slab, [i0[g], i1[g], th])
            row0 = pl.multiple_of(c * 64, 8)
            for a in range(8):
                pltpu.async_copy(
                    out3.at[a], p_hbm.at[pl.ds(row0 + 8 * a, 8), :], sem_out)
            for a in range(8):
                pltpu.make_async_copy(
                    out3.at[a], p_hbm.at[pl.ds(row0 + 8 * a, 8), :],
                    sem_out).wait()
            return carry

        lax.fori_loop(0, nchunk, body, 0)

    return transpose_k


@functools.lru_cache(maxsize=None)
def _make_pair_gather(n_rows: int, n_s_pad: int):
    rw = n_rows // _NW
    mesh = plsc.VectorSubcoreMesh(core_axis_name="c", subcore_axis_name="s")

    @functools.partial(
        pl.kernel,
        mesh=mesh,
        compiler_params=pltpu.CompilerParams(use_tc_tiling_on_sc=False,
                                             needs_layout_passes=False),
        out_type=(
            jax.ShapeDtypeStruct((n_rows, 128), jnp.float32),
            jax.ShapeDtypeStruct((n_rows, 128), jnp.float32),
            jax.ShapeDtypeStruct((n_s_pad, 128), jnp.float32),
        ),
        scratch_types=[
            pltpu.VMEM((512, 2), jnp.int32),
            pltpu.VMEM((512,), jnp.int32),
            pltpu.VMEM((512,), jnp.int32),
            pltpu.VMEM((512, 128), jnp.float32),
            pltpu.VMEM((n_s_pad,), jnp.int32),
            pltpu.VMEM((n_s_pad, 128), jnp.float32),
            pltpu.SemaphoreType.DMA,
            pltpu.SemaphoreType.DMA,
        ],
    )
    def gather_k(qpair_hbm, spair_hbm, p_hbm, oute_hbm, outo_hbm, outs_hbm,
                 idx2_v, idxe_v, idxo_v, rows_v, idxs_v, rows_s, sem, sem2):
        wid = lax.axis_index("s") * _NC + lax.axis_index("c")
        lane = lax.iota(jnp.int32, 16)

        @pl.when(wid == 0)
        def _():
            pltpu.sync_copy(spair_hbm, idx2_v.at[pl.ds(0, FEW)])
            row = jnp.minimum(lane >> 1, FEW - 1)
            col = lane & 1
            vals = plsc.load_gather(idx2_v, [row, col])
            idxs_v[...] = jnp.where(lane < 2 * FEW, vals, 0)
            pltpu.async_copy(p_hbm.at[idxs_v], rows_s, sem2).wait()
            pltpu.sync_copy(rows_s, outs_hbm)

        nh = rw // 512
        for h in range(nh):
            base = wid * rw + h * 512
            pltpu.sync_copy(qpair_hbm.at[pl.ds(base, 512)], idx2_v)

            def flatten_body(k, carry):
                r = k * 16 + lane
                idxe_v[pl.ds(k * 16, 16)] = plsc.load_gather(
                    idx2_v, [r, r & 0])
                idxo_v[pl.ds(k * 16, 16)] = plsc.load_gather(
                    idx2_v, [r, (r & 0) + 1])
                return carry

            lax.fori_loop(0, 32, flatten_body, 0)
            pltpu.async_copy(p_hbm.at[idxe_v], rows_v, sem).wait()
            pltpu.sync_copy(rows_v, oute_hbm.at[pl.ds(base, 512)])
            pltpu.async_copy(p_hbm.at[idxo_v], rows_v, sem).wait()
            pltpu.sync_copy(rows_v, outo_hbm.at[pl.ds(base, 512)])

    return gather_k


def _dense_body(ev_ref, od_ref, par_ref, s_ref, W1T_ref, b1_ref, W2T_ref,
                b2_ref, g_ref, bb_ref, WihT_ref, bih_ref, WhhT_ref, bhh_ref,
                out_ref):
    f32 = jnp.float32
    ev = ev_ref[...]
    od = od_ref[...]
    par = par_ref[...]
    p0 = par[:, 0:1] > 0
    p1 = par[:, 1:2] > 0
    q = jnp.concatenate([
        jnp.where(p0, ev[:, EMBED_DIM:], ev[:, :EMBED_DIM]),
        jnp.where(p1, od[:, EMBED_DIM:], od[:, :EMBED_DIM]),
    ], axis=1)

    s = s_ref[...]
    h1 = jnp.maximum(
        jnp.dot(s, W1T_ref[...], preferred_element_type=f32) + b1_ref[...], 0.0)
    h2 = jnp.dot(h1, W2T_ref[...], preferred_element_type=f32) + b2_ref[...] + s
    mu = jnp.mean(h2, axis=-1, keepdims=True)
    var = jnp.mean((h2 - mu) ** 2, axis=-1, keepdims=True)
    sg = g_ref[...] * (h2 - mu) / (jnp.sqrt(var) + 1e-6) + bb_ref[...]

    col = lax.broadcasted_iota(jnp.int32, (1, SUP_PAD), 1)
    neg = jnp.where(col < FEW, 0.0, -1e30)

    bm = q.shape[0]
    xW = jnp.dot(q, WihT_ref[...], preferred_element_type=f32) + bih_ref[...]
    h_r = jnp.zeros((bm, HID), f32)
    c = jnp.zeros((bm, HID), f32)
    h = q
    for step in range(STEPS):
        gates = xW + bhh_ref[...]
        if step > 0:
            gates = gates + jnp.dot(h_r, WhhT_ref[...],
                                    preferred_element_type=f32)
        sig = lambda x: 0.5 * jnp.tanh(0.5 * x) + 0.5
        i_g = sig(gates[:, 0:HID])
        f_g = sig(gates[:, HID:2 * HID])
        g_g = jnp.tanh(gates[:, 2 * HID:3 * HID])
        o_g = sig(gates[:, 3 * HID:4 * HID])
        c = f_g * c + i_g * g_g
        h_new = o_g * jnp.tanh(c)
        h = q + h_new[:, :D_MODEL]
        logits = lax.dot_general(h, sg, (((1,), (1,)), ((), ())),
                                 preferred_element_type=f32) + neg
        attn = jax.nn.softmax(logits, axis=1)
        r = jnp.dot(attn, sg, preferred_element_type=f32)
        h_r = jnp.concatenate([h, r], axis=1)
    out_ref[...] = lax.dot_general(h, sg, (((1,), (1,)), ((), ())),
                                   preferred_element_type=f32)


def _dense_call(ev, od, par, s8, W1T, b1, W2T, b2, ln_g, ln_b, WihT, bih,
                WhhT, bhh, bm: int, interpret: bool = False):
    B = ev.shape[0]
    grid = (B // bm,)
    full = lambda shape: pl.BlockSpec(shape, lambda i: (0, 0))
    return pl.pallas_call(
        _dense_body,
        grid=grid,
        in_specs=[
            pl.BlockSpec((bm, 128), lambda i: (i, 0)),
            pl.BlockSpec((bm, 128), lambda i: (i, 0)),
            pl.BlockSpec((bm, 2), lambda i: (i, 0)),
            full((SUP_PAD, D_MODEL)),
            full((D_MODEL, D_INNER)),
            full((1, D_INNER)),
            full((D_INNER, D_MODEL)),
            full((1, D_MODEL)),
            full((1, D_MODEL)),
            full((1, D_MODEL)),
            full((D_MODEL, 4 * HID)),
            full((1, 4 * HID)),
            full((HID, 4 * HID)),
            full((1, 4 * HID)),
        ],
        out_specs=pl.BlockSpec((bm, SUP_PAD), lambda i: (i, 0)),
        out_shape=jax.ShapeDtypeStruct((B, SUP_PAD), jnp.float32),
        compiler_params=pltpu.CompilerParams(
            dimension_semantics=("arbitrary",)),
        interpret=interpret,
    )(ev, od, par, s8, W1T, b1, W2T, b2, ln_g, ln_b, WihT, bih, WhhT, bhh)


def kernel(query, support, emb, W1, b1, W2, b2, ln_g, ln_b, W_ih, W_hh,
           b_ih, b_hh):
    B = query.shape[0]
    p_table = _make_transpose()(emb.T)
    qpair = query >> 1
    spair = support >> 1
    par_q = query & 1
    par_s = support & 1
    rows_e, rows_o, rows_s = _make_pair_gather(B, 16)(qpair, spair, p_table)
    s2 = jnp.where((par_s > 0)[:, :, None], rows_s[:2 * FEW, 64:].reshape(
        FEW, 2, EMBED_DIM), rows_s[:2 * FEW, :64].reshape(FEW, 2, EMBED_DIM))
    s8 = jnp.concatenate([s2.reshape(FEW, D_MODEL),
                          jnp.zeros((SUP_PAD - FEW, D_MODEL), jnp.float32)],
                         axis=0)
    scores8 = _dense_call(
        rows_e, rows_o, par_q, s8, W1.T, b1[None, :], W2.T, b2[None, :],
        ln_g[None, :], ln_b[None, :], W_ih.T, b_ih[None, :], W_hh.T,
        b_hh[None, :], bm=2048)
    return scores8[:, :FEW]


# speedup vs baseline: 1.3236x; 1.3236x over previous
import functools

import jax
import jax.numpy as jnp
from jax import lax
from jax.experimental import pallas as pl
from jax.experimental.pallas import tpu as pltpu
from jax.experimental.pallas import tpu_sc as plsc

EMBED_DIM = 64
D_MODEL = 2 * EMBED_DIM
D_INNER = 2 * D_MODEL
HID = 2 * D_MODEL
STEPS = 4
FEW = 5
SUP_PAD = 8

_NC, _NS = 2, 16
_NW = _NC * _NS

_V = 1000001
_TCOLS = (_V + 127) // 128
_PROWS = _TCOLS * 64


@functools.lru_cache(maxsize=None)
def _make_transpose():
    mesh = plsc.VectorSubcoreMesh(core_axis_name="c", subcore_axis_name="s")

    @functools.partial(
        pl.kernel,
        mesh=mesh,
        compiler_params=pltpu.CompilerParams(use_tc_tiling_on_sc=True,
                                             needs_layout_passes=False,
                                             disable_bounds_checks=True),
        out_type=jax.ShapeDtypeStruct((_PROWS, 128), jnp.float32),
        scratch_types=[
            pltpu.VMEM((8, 8, 128), jnp.float32),
            pltpu.VMEM((8, 8, 128), jnp.float32),
            pltpu.VMEM((8, 8, 128), jnp.float32),
            pltpu.VMEM((8, 8, 128), jnp.float32),
            pltpu.SemaphoreType.DMA,
            pltpu.SemaphoreType.DMA,
            pltpu.SemaphoreType.DMA,
            pltpu.SemaphoreType.DMA,
        ],
    )
    def transpose_k(embT_hbm, p_hbm, slabA, slabB, outA, outB,
                    semA, semB, semoA, semoB):
        wid = lax.axis_index("s") * _NC + lax.axis_index("c")
        lane = lax.iota(jnp.int32, 16)
        npair = (_TCOLS + 2 * _NW - 1) // (2 * _NW)

        i0 = [(jnp.full((16,), (16 * g) & 63, jnp.int32) + lane) >> 3
              for g in range(8)]
        i1 = [(jnp.full((16,), (16 * g) & 63, jnp.int32) + lane) & 7
              for g in range(8)]

        def issue_in(c, slab, sem):
            col0 = pl.multiple_of(c * 128, 128)
            for a in range(8):
                pltpu.async_copy(
                    embT_hbm.at[pl.ds(8 * a, 8), pl.ds(col0, 128)],
                    slab.at[a], sem)

        def wait_in(c, slab, sem):
            col0 = pl.multiple_of(c * 128, 128)
            for a in range(8):
                pltpu.make_async_copy(
                    embT_hbm.at[pl.ds(8 * a, 8), pl.ds(col0, 128)],
                    slab.at[a], sem).wait()

        def trans(slab, out3):
            for jt in range(8):
                for jr in range(8):
                    j = 8 * jt + jr
                    t0 = jnp.full((16,), 2 * j, jnp.int32)
                    t1 = jnp.full((16,), 2 * j + 1, jnp.int32)
                    vals = [plsc.load_gather(
                        slab, [i0[g], i1[g], t0 if g < 4 else t1])
                        for g in range(8)]
                    for g in range(8):
                        out3[jt, jr, pl.ds(16 * g, 16)] = vals[g]

        def issue_out(c, out3, sem):
            row0 = pl.multiple_of(c * 64, 8)
            for a in range(8):
                pltpu.async_copy(
                    out3.at[a], p_hbm.at[pl.ds(row0 + 8 * a, 8), :], sem)

        def wait_out(c, out3, sem):
            row0 = pl.multiple_of(c * 64, 8)
            for a in range(8):
                pltpu.make_async_copy(
                    out3.at[a], p_hbm.at[pl.ds(row0 + 8 * a, 8), :],
                    sem).wait()

        def body(k, carry):
            cA = wid + (2 * k) * _NW
            cB = wid + (2 * k + 1) * _NW

            @pl.when(cA < _TCOLS)
            def _():
                issue_in(cA, slabA, semA)

            @pl.when(cB < _TCOLS)
            def _():
                issue_in(cB, slabB, semB)

            @pl.when(cA < _TCOLS)
            def _():
                wait_in(cA, slabA, semA)
                trans(slabA, outA)
                issue_out(cA, outA, semoA)

            @pl.when(cB < _TCOLS)
            def _():
                wait_in(cB, slabB, semB)
                trans(slabB, outB)
                issue_out(cB, outB, semoB)

            @pl.when(cA < _TCOLS)
            def _():
                wait_out(cA, outA, semoA)

            @pl.when(cB < _TCOLS)
            def _():
                wait_out(cB, outB, semoB)
            return carry

        lax.fori_loop(0, npair, body, 0)

    return transpose_k


@functools.lru_cache(maxsize=None)
def _make_pair_gather(n_rows: int, n_s_pad: int):
    rw = n_rows // _NW
    mesh = plsc.VectorSubcoreMesh(core_axis_name="c", subcore_axis_name="s")

    @functools.partial(
        pl.kernel,
        mesh=mesh,
        compiler_params=pltpu.CompilerParams(use_tc_tiling_on_sc=False,
                                             needs_layout_passes=False),
        out_type=(
            jax.ShapeDtypeStruct((n_rows, 128), jnp.float32),
            jax.ShapeDtypeStruct((n_rows, 128), jnp.float32),
            jax.ShapeDtypeStruct((n_s_pad, 128), jnp.float32),
        ),
        scratch_types=[
            pltpu.VMEM((512, 2), jnp.int32),
            pltpu.VMEM((512,), jnp.int32),
            pltpu.VMEM((512,), jnp.int32),
            pltpu.VMEM((512, 128), jnp.float32),
            pltpu.VMEM((n_s_pad,), jnp.int32),
            pltpu.VMEM((n_s_pad, 128), jnp.float32),
            pltpu.SemaphoreType.DMA,
            pltpu.SemaphoreType.DMA,
        ],
    )
    def gather_k(qpair_hbm, spair_hbm, p_hbm, oute_hbm, outo_hbm, outs_hbm,
                 idx2_v, idxe_v, idxo_v, rows_v, idxs_v, rows_s, sem, sem2):
        wid = lax.axis_index("s") * _NC + lax.axis_index("c")
        lane = lax.iota(jnp.int32, 16)

        @pl.when(wid == 0)
        def _():
            pltpu.sync_copy(spair_hbm, idx2_v.at[pl.ds(0, FEW)])
            row = jnp.minimum(lane >> 1, FEW - 1)
            col = lane & 1
            vals = plsc.load_gather(idx2_v, [row, col])
            idxs_v[...] = jnp.where(lane < 2 * FEW, vals, 0)
            pltpu.async_copy(p_hbm.at[idxs_v], rows_s, sem2).wait()
            pltpu.sync_copy(rows_s, outs_hbm)

        nh = rw // 512
        for h in range(nh):
            base = wid * rw + h * 512
            pltpu.sync_copy(qpair_hbm.at[pl.ds(base, 512)], idx2_v)

            def flatten_body(k, carry):
                r = k * 16 + lane
                idxe_v[pl.ds(k * 16, 16)] = plsc.load_gather(
                    idx2_v, [r, r & 0])
                idxo_v[pl.ds(k * 16, 16)] = plsc.load_gather(
                    idx2_v, [r, (r & 0) + 1])
                return carry

            lax.fori_loop(0, 32, flatten_body, 0)
            pltpu.async_copy(p_hbm.at[idxe_v], rows_v, sem).wait()
            pltpu.sync_copy(rows_v, oute_hbm.at[pl.ds(base, 512)])
            pltpu.async_copy(p_hbm.at[idxo_v], rows_v, sem).wait()
            pltpu.sync_copy(rows_v, outo_hbm.at[pl.ds(base, 512)])

    return gather_k


def _dense_body(ev_ref, od_ref, par_ref, s_ref, W1T_ref, b1_ref, W2T_ref,
                b2_ref, g_ref, bb_ref, WihT_ref, bih_ref, WhhT_ref, bhh_ref,
                out_ref):
    f32 = jnp.float32
    ev = ev_ref[...]
    od = od_ref[...]
    par = par_ref[...]
    p0 = par[:, 0:1] > 0
    p1 = par[:, 1:2] > 0
    q = jnp.concatenate([
        jnp.where(p0, ev[:, EMBED_DIM:], ev[:, :EMBED_DIM]),
        jnp.where(p1, od[:, EMBED_DIM:], od[:, :EMBED_DIM]),
    ], axis=1)

    s = s_ref[...]
    h1 = jnp.maximum(
        jnp.dot(s, W1T_ref[...], preferred_element_type=f32) + b1_ref[...], 0.0)
    h2 = jnp.dot(h1, W2T_ref[...], preferred_element_type=f32) + b2_ref[...] + s
    mu = jnp.mean(h2, axis=-1, keepdims=True)
    var = jnp.mean((h2 - mu) ** 2, axis=-1, keepdims=True)
    sg = g_ref[...] * (h2 - mu) / (jnp.sqrt(var) + 1e-6) + bb_ref[...]

    col = lax.broadcasted_iota(jnp.int32, (1, SUP_PAD), 1)
    neg = jnp.where(col < FEW, 0.0, -1e30)

    bm = q.shape[0]
    xW = jnp.dot(q, WihT_ref[...], preferred_element_type=f32) + bih_ref[...]
    h_r = jnp.zeros((bm, HID), f32)
    c = jnp.zeros((bm, HID), f32)
    h = q
    for step in range(STEPS):
        gates = xW + bhh_ref[...]
        if step > 0:
            gates = gates + jnp.dot(h_r, WhhT_ref[...],
                                    preferred_element_type=f32)
        sig = lambda x: 0.5 * jnp.tanh(0.5 * x) + 0.5
        i_g = sig(gates[:, 0:HID])
        f_g = sig(gates[:, HID:2 * HID])
        g_g = jnp.tanh(gates[:, 2 * HID:3 * HID])
        o_g = sig(gates[:, 3 * HID:4 * HID])
        c = f_g * c + i_g * g_g
        h_new = o_g * jnp.tanh(c)
        h = q + h_new[:, :D_MODEL]
        logits = lax.dot_general(h, sg, (((1,), (1,)), ((), ())),
                                 preferred_element_type=f32) + neg
        attn = jax.nn.softmax(logits, axis=1)
        r = jnp.dot(attn, sg, preferred_element_type=f32)
        h_r = jnp.concatenate([h, r], axis=1)
    out_ref[...] = lax.dot_general(h, sg, (((1,), (1,)), ((), ())),
                                   preferred_element_type=f32)


def _dense_call(ev, od, par, s8, W1T, b1, W2T, b2, ln_g, ln_b, WihT, bih,
                WhhT, bhh, bm: int, interpret: bool = False):
    B = ev.shape[0]
    grid = (B // bm,)
    full = lambda shape: pl.BlockSpec(shape, lambda i: (0, 0))
    return pl.pallas_call(
        _dense_body,
        grid=grid,
        in_specs=[
            pl.BlockSpec((bm, 128), lambda i: (i, 0)),
            pl.BlockSpec((bm, 128), lambda i: (i, 0)),
            pl.BlockSpec((bm, 2), lambda i: (i, 0)),
            full((SUP_PAD, D_MODEL)),
            full((D_MODEL, D_INNER)),
            full((1, D_INNER)),
            full((D_INNER, D_MODEL)),
            full((1, D_MODEL)),
            full((1, D_MODEL)),
            full((1, D_MODEL)),
            full((D_MODEL, 4 * HID)),
            full((1, 4 * HID)),
            full((HID, 4 * HID)),
            full((1, 4 * HID)),
        ],
        out_specs=pl.BlockSpec((bm, SUP_PAD), lambda i: (i, 0)),
        out_shape=jax.ShapeDtypeStruct((B, SUP_PAD), jnp.float32),
        compiler_params=pltpu.CompilerParams(
            dimension_semantics=("arbitrary",)),
        interpret=interpret,
    )(ev, od, par, s8, W1T, b1, W2T, b2, ln_g, ln_b, WihT, bih, WhhT, bhh)


def kernel(query, support, emb, W1, b1, W2, b2, ln_g, ln_b, W_ih, W_hh,
           b_ih, b_hh):
    B = query.shape[0]
    p_table = _make_transpose()(emb.T)
    qpair = query >> 1
    spair = support >> 1
    par_q = query & 1
    par_s = support & 1
    rows_e, rows_o, rows_s = _make_pair_gather(B, 16)(qpair, spair, p_table)
    s2 = jnp.where((par_s > 0)[:, :, None], rows_s[:2 * FEW, 64:].reshape(
        FEW, 2, EMBED_DIM), rows_s[:2 * FEW, :64].reshape(FEW, 2, EMBED_DIM))
    s8 = jnp.concatenate([s2.reshape(FEW, D_MODEL),
                          jnp.zeros((SUP_PAD - FEW, D_MODEL), jnp.float32)],
                         axis=0)
    scores8 = _dense_call(
        rows_e, rows_o, par_q, s8, W1.T, b1[None, :], W2.T, b2[None, :],
        ln_g[None, :], ln_b[None, :], W_ih.T, b_ih[None, :], W_hh.T,
        b_hh[None, :], bm=2048)
    return scores8[:, :FEW]


# speedup vs baseline: 2.8523x; 2.1549x over previous
import functools

import jax
import jax.numpy as jnp
from jax import lax
from jax.experimental import pallas as pl
from jax.experimental.pallas import tpu as pltpu
from jax.experimental.pallas import tpu_sc as plsc

EMBED_DIM = 64
D_MODEL = 2 * EMBED_DIM
D_INNER = 2 * D_MODEL
HID = 2 * D_MODEL
STEPS = 4
FEW = 5
SUP_PAD = 8

_NC, _NS = 2, 16
_NW = _NC * _NS


@functools.lru_cache(maxsize=None)
def _make_sc_gather(n_q_flat: int, n_s_pad: int):
    n_per_w = n_q_flat // _NW
    mesh = plsc.VectorSubcoreMesh(core_axis_name="c", subcore_axis_name="s")

    n_rows_w = n_per_w // 2

    @functools.partial(
        pl.kernel,
        mesh=mesh,
        compiler_params=pltpu.CompilerParams(use_tc_tiling_on_sc=False,
                                             needs_layout_passes=False),
        out_type=(
            jax.ShapeDtypeStruct((n_q_flat, EMBED_DIM), jnp.float32),
            jax.ShapeDtypeStruct((n_s_pad, EMBED_DIM), jnp.float32),
        ),
        scratch_types=[
            pltpu.VMEM((n_per_w // 2, 2), jnp.int32),
            pltpu.VMEM((n_per_w,), jnp.int32),
            pltpu.VMEM((n_per_w, EMBED_DIM), jnp.float32),
            pltpu.VMEM((FEW, 2), jnp.int32),
            pltpu.VMEM((n_s_pad,), jnp.int32),
            pltpu.VMEM((n_s_pad, EMBED_DIM), jnp.float32),
            pltpu.SemaphoreType.DMA,
            pltpu.SemaphoreType.DMA,
        ],
    )
    def gather_k(query_hbm, support_hbm, table_hbm, outq_hbm, outs_hbm,
                 idx2_v, idx_v, rows_v, idxs2_v, idxs_v, rows_s, sem, sem2):
        wid = lax.axis_index("s") * _NC + lax.axis_index("c")
        base = wid * n_per_w
        pltpu.sync_copy(query_hbm.at[pl.ds(wid * n_rows_w, n_rows_w)], idx2_v)
        lane = lax.iota(jnp.int32, 16)

        def flatten_body(k, carry):
            g = k * 16 + lane
            row = lax.shift_right_logical(g, 1)
            col = lax.bitwise_and(g, 1)
            idx_v[pl.ds(k * 16, 16)] = plsc.load_gather(idx2_v, [row, col])
            return carry

        lax.fori_loop(0, n_per_w // 16, flatten_body, 0)
        cp = pltpu.async_copy(table_hbm.at[idx_v], rows_v, sem)

        @pl.when(wid == 0)
        def _():
            pltpu.sync_copy(support_hbm, idxs2_v)
            g = lane
            row = jnp.minimum(lax.shift_right_logical(g, 1), FEW - 1)
            col = lax.bitwise_and(g, 1)
            vals = plsc.load_gather(idxs2_v, [row, col])
            idxs_v[...] = jnp.where(lane < 2 * FEW, vals, 0)
            pltpu.async_copy(table_hbm.at[idxs_v], rows_s, sem2).wait()
            pltpu.sync_copy(rows_s, outs_hbm)

        cp.wait()
        pltpu.sync_copy(rows_v, outq_hbm.at[pl.ds(base, n_per_w)])

    return gather_k


def _dense_body(q_ref, s_ref, W1T_ref, b1_ref, W2T_ref, b2_ref, g_ref,
                bb_ref, WihT_ref, bih_ref, WhhT_ref, bhh_ref, out_ref):
    f32 = jnp.float32
    s = s_ref[...]
    h1 = jnp.maximum(
        jnp.dot(s, W1T_ref[...], preferred_element_type=f32) + b1_ref[...], 0.0)
    h2 = jnp.dot(h1, W2T_ref[...], preferred_element_type=f32) + b2_ref[...] + s
    mu = jnp.mean(h2, axis=-1, keepdims=True)
    var = jnp.mean((h2 - mu) ** 2, axis=-1, keepdims=True)
    sg = g_ref[...] * (h2 - mu) / (jnp.sqrt(var) + 1e-6) + bb_ref[...]

    col = lax.broadcasted_iota(jnp.int32, (1, SUP_PAD), 1)
    neg = jnp.where(col < FEW, 0.0, -1e30)

    q = q_ref[...]
    bm = q.shape[0]
    xW = jnp.dot(q, WihT_ref[...], preferred_element_type=f32) + bih_ref[...]
    h_r = jnp.zeros((bm, HID), f32)
    c = jnp.zeros((bm, HID), f32)
    h = q
    for step in range(STEPS):
        gates = xW + bhh_ref[...]
        if step > 0:
            gates = gates + jnp.dot(h_r, WhhT_ref[...],
                                    preferred_element_type=f32)
        sig = lambda x: 0.5 * jnp.tanh(0.5 * x) + 0.5
        i_g = sig(gates[:, 0:HID])
        f_g = sig(gates[:, HID:2 * HID])
        g_g = jnp.tanh(gates[:, 2 * HID:3 * HID])
        o_g = sig(gates[:, 3 * HID:4 * HID])
        c = f_g * c + i_g * g_g
        h_new = o_g * jnp.tanh(c)
        h = q + h_new[:, :D_MODEL]
        logits = lax.dot_general(h, sg, (((1,), (1,)), ((), ())),
                                 preferred_element_type=f32) + neg
        attn = jax.nn.softmax(logits, axis=1)
        r = jnp.dot(attn, sg, preferred_element_type=f32)
        h_r = jnp.concatenate([h, r], axis=1)
    out_ref[...] = lax.dot_general(h, sg, (((1,), (1,)), ((), ())),
                                   preferred_element_type=f32)


def _dense_call(q, s8, W1T, b1, W2T, b2, ln_g, ln_b, WihT, bih, WhhT, bhh,
                bm: int, interpret: bool = False):
    B = q.shape[0]
    grid = (B // bm,)
    full = lambda shape: pl.BlockSpec(shape, lambda i: (0, 0))
    return pl.pallas_call(
        _dense_body,
        grid=grid,
        in_specs=[
            pl.BlockSpec((bm, D_MODEL), lambda i: (i, 0)),
            full((SUP_PAD, D_MODEL)),
            full((D_MODEL, D_INNER)),
            full((1, D_INNER)),
            full((D_INNER, D_MODEL)),
            full((1, D_MODEL)),
            full((1, D_MODEL)),
            full((1, D_MODEL)),
            full((D_MODEL, 4 * HID)),
            full((1, 4 * HID)),
            full((HID, 4 * HID)),
            full((1, 4 * HID)),
        ],
        out_specs=pl.BlockSpec((bm, SUP_PAD), lambda i: (i, 0)),
        out_shape=jax.ShapeDtypeStruct((B, SUP_PAD), jnp.float32),
        compiler_params=pltpu.CompilerParams(
            dimension_semantics=("arbitrary",)),
        interpret=interpret,
    )(q, s8, W1T, b1, W2T, b2, ln_g, ln_b, WihT, bih, WhhT, bhh)


def kernel(query, support, emb, W1, b1, W2, b2, ln_g, ln_b, W_ih, W_hh,
           b_ih, b_hh):
    B = query.shape[0]
    n_q = B * 2
    rows_q, rows_s = _make_sc_gather(n_q, 16)(query, support, emb)
    q = rows_q.reshape(B, D_MODEL)
    s = rows_s[:2 * FEW].reshape(FEW, D_MODEL)
    s8 = jnp.concatenate([s, jnp.zeros((SUP_PAD - FEW, D_MODEL),
                                       jnp.float32)], axis=0)
    scores8 = _dense_call(
        q, s8, W1.T, b1[None, :], W2.T, b2[None, :], ln_g[None, :],
        ln_b[None, :], W_ih.T, b_ih[None, :], W_hh.T, b_hh[None, :], bm=2048)
    return scores8[:, :FEW]
